# Initial kernel scaffold; baseline (speedup 1.0000x reference)
import jax
import jax.numpy as jnp
from jax.experimental import pallas as pl


def kernel(query, support, edge_indices, features, K_points, K_values):
    raise NotImplementedError("write your pallas kernel here")


# speedup vs baseline: 2.2851x; 2.2851x over previous
import functools

import jax
import jax.numpy as jnp
from jax import lax
from jax.experimental import pallas as pl
from jax.experimental.pallas import tpu as pltpu
from jax.experimental.pallas import tpu_sc as plsc

N = 10000
NN = 32
IN_C = 128
OUT_C = 128
K = 15
KP_EXTENT = 0.1

E = N * NN
POSW = 16

B = 80
G = B // 8
EPB = B * NN
NBLK = N // B

NC = 2
NS = 16
NW = NC * NS
PER_W = E // NW
CH = 80
NCHUNK = PER_W // CH


def _sc_gather(features, support_pad, idx_flat):
    mesh = plsc.VectorSubcoreMesh(core_axis_name="c", subcore_axis_name="s")

    @functools.partial(
        pl.kernel,
        mesh=mesh,
        out_type=[
            jax.ShapeDtypeStruct((E, IN_C), jnp.float32),
            jax.ShapeDtypeStruct((E, POSW), jnp.float32),
        ],
        scratch_types=[
            pltpu.VMEM((CH,), jnp.int32),
            pltpu.VMEM((CH, IN_C), jnp.float32),
            pltpu.VMEM((CH, POSW), jnp.float32),
            pltpu.SemaphoreType.DMA,
            pltpu.SemaphoreType.DMA,
        ],
        compiler_params=pltpu.CompilerParams(use_tc_tiling_on_sc=False),
    )
    def gather_kernel(feat_hbm, pos_hbm, idx_hbm, nf_hbm, npos_hbm,
                      idx_v, feat_v, pos_v, sem_f, sem_p):
        wid = lax.axis_index("s") * NC + lax.axis_index("c")
        base = wid * PER_W

        def body(c, carry):
            off = base + c * CH
            pltpu.sync_copy(idx_hbm.at[pl.ds(off, CH)], idx_v)
            cp_f = pltpu.async_copy(feat_hbm.at[idx_v], feat_v, sem_f)
            cp_p = pltpu.async_copy(pos_hbm.at[idx_v], pos_v, sem_p)
            cp_f.wait()
            cp_p.wait()
            pltpu.sync_copy(feat_v, nf_hbm.at[pl.ds(off, CH)])
            pltpu.sync_copy(pos_v, npos_hbm.at[pl.ds(off, CH)])
            return carry

        lax.fori_loop(0, NCHUNK, body, 0)

    return gather_kernel(features, support_pad, idx_flat)


def _tc_body(nf_ref, npos_ref, q_ref, kpt_ref, kv_ref, out_ref, wf_ref):
    f32 = jnp.float32
    q = q_ref[...]
    qe = jnp.broadcast_to(q[:, None, :], (B, NN, 4)).reshape(EPB, 4)
    npos = npos_ref[...]
    sq = jnp.zeros((EPB, 16), f32)
    for d in range(3):
        dd = (npos[:, d:d + 1] - qe[:, d:d + 1]) - kpt_ref[d:d + 1, :]
        sq = sq + dd * dd
    w_t = jnp.maximum(1.0 - jnp.sqrt(sq) * (1.0 / KP_EXTENT), 0.0)

    ri = lax.broadcasted_iota(jnp.int32, (128, 2 * 128), 0)
    ci = lax.broadcasted_iota(jnp.int32, (128, 2 * 128), 1)
    mask = ((ri % 8) == (ci // NN)).astype(f32)
    ri2 = lax.broadcasted_iota(jnp.int32, (128, 16), 0)
    ki = lax.broadcasted_iota(jnp.int32, (128, 16), 1)
    ksel = (ki == (ri2 // 8)).astype(f32)

    for g in range(G):
        w_g = w_t[256 * g:256 * (g + 1), :]
        bdw = lax.dot_general(ksel, w_g, (((1,), (1,)), ((), ())),
                              preferred_element_type=f32)
        bd = bdw * mask
        nf_g = nf_ref[pl.ds(256 * g, 256), :]
        wf_g = jnp.dot(bd, nf_g, preferred_element_type=f32)
        wf_ref[:, 8 * g:8 * (g + 1), :] = wf_g.reshape(16, 8, 128)

    acc = jnp.zeros((B, OUT_C), f32)
    for k in range(16):
        acc = acc + jnp.dot(wf_ref[k], kv_ref[k], preferred_element_type=f32)
    out_ref[...] = acc


def kernel(query, support, edge_indices, features, K_points, K_values):
    f32 = jnp.float32
    idx_flat = edge_indices.reshape(-1).astype(jnp.int32)
    support_pad = jnp.pad(support.astype(f32), ((0, 0), (0, POSW - 3)))
    nf, npos = _sc_gather(features.astype(f32), support_pad, idx_flat)

    q4 = jnp.pad(query.astype(f32), ((0, 0), (0, 1)))
    kpt = jnp.transpose(
        jnp.pad(K_points.astype(f32), ((0, 1), (0, 0)),
                constant_values=1e6))
    kpt = jnp.pad(kpt, ((0, 5), (0, 0)))
    kv = jnp.pad(K_values.astype(f32), ((0, 1), (0, 0), (0, 0)))

    out = pl.pallas_call(
        _tc_body,
        grid=(NBLK,),
        in_specs=[
            pl.BlockSpec((EPB, IN_C), lambda i: (i, 0)),
            pl.BlockSpec((EPB, POSW), lambda i: (i, 0)),
            pl.BlockSpec((B, 4), lambda i: (i, 0)),
            pl.BlockSpec((8, 16), lambda i: (0, 0)),
            pl.BlockSpec((16, 128, 128), lambda i: (0, 0, 0)),
        ],
        out_specs=pl.BlockSpec((B, OUT_C), lambda i: (i, 0)),
        out_shape=jax.ShapeDtypeStruct((N, OUT_C), f32),
        scratch_shapes=[pltpu.VMEM((16, B, 128), f32)],
    )(nf, npos, q4, kpt, kv)
    return out


# speedup vs baseline: 3.1008x; 1.3570x over previous
import functools

import jax
import jax.numpy as jnp
from jax import lax
from jax.experimental import pallas as pl
from jax.experimental.pallas import tpu as pltpu
from jax.experimental.pallas import tpu_sc as plsc

N = 10000
NN = 32
IN_C = 128
OUT_C = 128
K = 15
KP_EXTENT = 0.1

E = N * NN
POSW = 16

B = 200
G = B // 8
EPB = B * NN
NBLK = N // B

NC = 2
NS = 16
NW = NC * NS
PER_W = E // NW
CH = 125
NCHUNK = PER_W // CH
NBUF = 4
NSUP = NCHUNK // NBUF


def _sc_gather(features, support_pad, idx_grp):
    mesh = plsc.VectorSubcoreMesh(core_axis_name="c", subcore_axis_name="s")
    fdt = features.dtype

    @functools.partial(
        pl.kernel,
        mesh=mesh,
        out_type=[
            jax.ShapeDtypeStruct((E, IN_C), fdt),
            jax.ShapeDtypeStruct((E, POSW), jnp.float32),
        ],
        scratch_types=(
            [pltpu.VMEM((NCHUNK, CH), jnp.int32)]
            + [pltpu.VMEM((CH, IN_C), fdt) for _ in range(NBUF)]
            + [pltpu.VMEM((CH, POSW), jnp.float32) for _ in range(NBUF)]
            + [pltpu.SemaphoreType.DMA] * (2 * NBUF)
        ),
        compiler_params=pltpu.CompilerParams(use_tc_tiling_on_sc=False),
    )
    def gather_kernel(feat_hbm, pos_hbm, idx_hbm, nf_hbm, npos_hbm,
                      idx_all, *bufs):
        feat_v = bufs[:NBUF]
        pos_v = bufs[NBUF:2 * NBUF]
        gsem = bufs[2 * NBUF:2 * NBUF + NBUF]
        wsem = bufs[2 * NBUF + NBUF:]
        wid = lax.axis_index("s") * NC + lax.axis_index("c")
        base = wid * PER_W

        pltpu.sync_copy(idx_hbm.at[wid], idx_all)

        def start_g(c, j):
            pltpu.async_copy(feat_hbm.at[idx_all.at[c]], feat_v[j], gsem[j])
            pltpu.async_copy(pos_hbm.at[idx_all.at[c]], pos_v[j], gsem[j])

        def wait_g(j):
            pltpu.make_async_copy(feat_hbm.at[pl.ds(0, CH)], feat_v[j],
                                  gsem[j]).wait()
            pltpu.make_async_copy(pos_hbm.at[pl.ds(0, CH)], pos_v[j],
                                  gsem[j]).wait()

        def start_w(c, j):
            off = base + c * CH
            pltpu.async_copy(feat_v[j], nf_hbm.at[pl.ds(off, CH)], wsem[j])
            pltpu.async_copy(pos_v[j], npos_hbm.at[pl.ds(off, CH)], wsem[j])

        def wait_w(j):
            pltpu.make_async_copy(feat_v[j], nf_hbm.at[pl.ds(0, CH)],
                                  wsem[j]).wait()
            pltpu.make_async_copy(pos_v[j], npos_hbm.at[pl.ds(0, CH)],
                                  wsem[j]).wait()

        for j in range(NBUF):
            start_g(j, j)

        def body(cc, carry):
            for j in range(NBUF):
                c = cc * NBUF + j
                wait_g(j)
                start_w(c, j)

                @pl.when(cc < NSUP - 1)
                def _():
                    wait_w(j)
                    start_g(c + NBUF, j)

            return carry

        lax.fori_loop(0, NSUP, body, 0)
        for j in range(NBUF):
            wait_w(j)

    return gather_kernel(features, support_pad, idx_grp)


def _tc_body(nf_ref, npos_ref, q_ref, kpt_ref, kv_ref, out_ref, wf_ref):
    f32 = jnp.float32
    q = q_ref[...]
    qe = jnp.broadcast_to(q[:, None, :], (B, NN, 4)).reshape(EPB, 4)
    npos = npos_ref[...]
    sq = jnp.zeros((EPB, 16), f32)
    for d in range(3):
        dd = (npos[:, d:d + 1] - qe[:, d:d + 1]) - kpt_ref[d:d + 1, :]
        sq = sq + dd * dd
    w_t = jnp.maximum(1.0 - jnp.sqrt(sq) * (1.0 / KP_EXTENT), 0.0)

    ri = lax.broadcasted_iota(jnp.int32, (128, 2 * 128), 0)
    ci = lax.broadcasted_iota(jnp.int32, (128, 2 * 128), 1)
    mask = ((ri % 8) == (ci // NN)).astype(f32)
    ri2 = lax.broadcasted_iota(jnp.int32, (128, 16), 0)
    ki = lax.broadcasted_iota(jnp.int32, (128, 16), 1)
    ksel = (ki == (ri2 // 8)).astype(f32)

    for g in range(G):
        w_g = w_t[256 * g:256 * (g + 1), :]
        bdw = lax.dot_general(ksel, w_g, (((1,), (1,)), ((), ())),
                              preferred_element_type=f32)
        bd = bdw * mask
        nf_g = nf_ref[pl.ds(256 * g, 256), :]
        wf_g = jnp.dot(bd, nf_g, preferred_element_type=f32)
        wf_ref[:, 8 * g:8 * (g + 1), :] = wf_g.reshape(16, 8, 128)

    acc = jnp.zeros((B, OUT_C), f32)
    for k in range(16):
        acc = acc + jnp.dot(wf_ref[k], kv_ref[k], preferred_element_type=f32)
    out_ref[...] = acc


def kernel(query, support, edge_indices, features, K_points, K_values):
    f32 = jnp.float32
    idx_grp = edge_indices.astype(jnp.int32).reshape(NW, NCHUNK, CH)
    support_pad = jnp.pad(support.astype(f32), ((0, 0), (0, POSW - 3)))
    nf, npos = _sc_gather(features.astype(f32), support_pad, idx_grp)

    q4 = jnp.pad(query.astype(f32), ((0, 0), (0, 1)))
    kpt = jnp.transpose(
        jnp.pad(K_points.astype(f32), ((0, 1), (0, 0)),
                constant_values=1e6))
    kpt = jnp.pad(kpt, ((0, 5), (0, 0)))
    kv = jnp.pad(K_values.astype(f32), ((0, 1), (0, 0), (0, 0)))

    out = pl.pallas_call(
        _tc_body,
        grid=(NBLK,),
        in_specs=[
            pl.BlockSpec((EPB, IN_C), lambda i: (i, 0)),
            pl.BlockSpec((EPB, POSW), lambda i: (i, 0)),
            pl.BlockSpec((B, 4), lambda i: (i, 0)),
            pl.BlockSpec((8, 16), lambda i: (0, 0)),
            pl.BlockSpec((16, 128, 128), lambda i: (0, 0, 0)),
        ],
        out_specs=pl.BlockSpec((B, OUT_C), lambda i: (i, 0)),
        out_shape=jax.ShapeDtypeStruct((N, OUT_C), f32),
        scratch_shapes=[pltpu.VMEM((16, B, 128), f32)],
    )(nf, npos, q4, kpt, kv)
    return out
